# Initial kernel scaffold; baseline (speedup 1.0000x reference)
import jax
import jax.numpy as jnp
from jax.experimental import pallas as pl


def kernel(x_i, x_j, params, edge_index_i, edge_index_j, batch_i, batch_j):
    raise NotImplementedError("write your pallas kernel here")


# speedup vs baseline: 2.9199x; 2.9199x over previous
import functools

import jax
import jax.numpy as jnp
from jax import lax
from jax.experimental import pallas as pl
from jax.experimental.pallas import tpu as pltpu
from jax.experimental.pallas import tpu_sc as plsc

N = 10000
E = 320000
G = 256
NC = 2
NT = 16
CH = 128
C = 160
TOT = NT * C * CH
ACC_ROWS = 10240
ZROWS = ACC_ROWS // (NT * CH)


@functools.lru_cache(maxsize=None)
def _make_agg(D):
    mesh = plsc.VectorSubcoreMesh(
        core_axis_name="c", subcore_axis_name="s", num_cores=NC,
        num_subcores=NT)

    @functools.partial(
        pl.kernel,
        mesh=mesh,
        compiler_params=pltpu.CompilerParams(use_tc_tiling_on_sc=False),
        out_type=jax.ShapeDtypeStruct((2 * ACC_ROWS, D), jnp.float32),
        scratch_types=[
            pltpu.VMEM((C, CH), jnp.int32),
            pltpu.VMEM((C, CH), jnp.int32),
            pltpu.VMEM((CH, D), jnp.float32),
            pltpu.VMEM((CH, D), jnp.float32),
            pltpu.VMEM((CH, D), jnp.float32),
            pltpu.VMEM_SHARED((ACC_ROWS, D), jnp.float32),
            pltpu.SemaphoreType.DMA,
            pltpu.SemaphoreType.DMA,
        ],
    )
    def agg(x_hbm, src_hbm, dst_hbm, zrow_hbm, out_hbm,
            src_v, dst_v, bufa, bufb, zbuf, acc, sema, semb):
        c = lax.axis_index("c")
        s = lax.axis_index("s")
        row0 = (c * NT + s) * C
        pltpu.sync_copy(src_hbm.at[pl.ds(row0, C)], src_v)
        pltpu.sync_copy(dst_hbm.at[pl.ds(row0, C)], dst_v)
        pltpu.sync_copy(zrow_hbm, zbuf)
        for t in range(ZROWS):
            pltpu.sync_copy(zbuf, acc.at[pl.ds((s * ZROWS + t) * CH, CH)])
        plsc.subcore_barrier()

        def step(k, carry):
            ga = pltpu.async_copy(x_hbm.at[src_v.at[2 * k]], bufa, sema)
            gb = pltpu.async_copy(x_hbm.at[src_v.at[2 * k + 1]], bufb, semb)
            ga.wait()
            pltpu.sync_copy(bufa, acc.at[dst_v.at[2 * k]], add=True)
            gb.wait()
            pltpu.sync_copy(bufb, acc.at[dst_v.at[2 * k + 1]], add=True)
            return carry

        lax.fori_loop(0, C // 2, step, 0)
        plsc.subcore_barrier()
        for t in range(ZROWS):
            r0 = (s * ZROWS + t) * CH
            pltpu.sync_copy(acc.at[pl.ds(r0, CH)], bufa)
            pltpu.sync_copy(bufa, out_hbm.at[pl.ds(c * ACC_ROWS + r0, CH)])

    return agg


def _dense_layer(x, aggs, wa, ba, wb, bb, gg, be, scale):
    _, _, din = x.shape
    dh = wa.shape[0]
    dout = wb.shape[0]
    na = len(aggs)

    def body(x_ref, *refs):
        a_refs = refs[:na]
        (wa_ref, ba_ref, wb_ref, bb_ref, g_ref, be_ref, sc_ref,
         o_ref) = refs[na:]
        agg = jnp.concatenate([a[0] for a in a_refs], axis=1)
        h = sc_ref[...] * x_ref[0] + agg
        h = lax.dot_general(h, wa_ref[...], (((1,), (1,)), ((), ())),
                            preferred_element_type=jnp.float32) + ba_ref[...]
        h = jnp.maximum(h, 0.0)
        h = lax.dot_general(h, wb_ref[...], (((1,), (1,)), ((), ())),
                            preferred_element_type=jnp.float32) + bb_ref[...]
        m = jnp.mean(h, axis=0, keepdims=True)
        d = h - m
        v = jnp.mean(d * d, axis=0, keepdims=True)
        o_ref[0] = g_ref[...] * d * lax.rsqrt(v + 1e-5) + be_ref[...]

    return pl.pallas_call(
        body,
        grid=(2,),
        in_specs=[
            pl.BlockSpec((1, N, din), lambda i: (i, 0, 0)),
        ] + [
            pl.BlockSpec((1, N, a.shape[2]), lambda i: (i, 0, 0))
            for a in aggs
        ] + [
            pl.BlockSpec((dh, din), lambda i: (0, 0)),
            pl.BlockSpec((1, dh), lambda i: (0, 0)),
            pl.BlockSpec((dout, dh), lambda i: (0, 0)),
            pl.BlockSpec((1, dout), lambda i: (0, 0)),
            pl.BlockSpec((1, dout), lambda i: (0, 0)),
            pl.BlockSpec((1, dout), lambda i: (0, 0)),
            pl.BlockSpec((1, din), lambda i: (0, 0)),
        ],
        out_specs=pl.BlockSpec((1, N, dout), lambda i: (i, 0, 0)),
        out_shape=jax.ShapeDtypeStruct((2, N, dout), jnp.float32),
    )(x, *aggs, wa, ba, wb, bb, gg, be, scale)


BB = 2000
NBLK = N // BB


def _pool_sums(x3, batch):
    def body(x_ref, b_ref, s_ref, c_ref):
        j = pl.program_id(1)

        @pl.when(j == 0)
        def _():
            s_ref[...] = jnp.zeros_like(s_ref)
            c_ref[...] = jnp.zeros_like(c_ref)

        oh = (lax.broadcasted_iota(jnp.int32, (BB, G), 1)
              == b_ref[0]).astype(jnp.float32)
        s_ref[0] += lax.dot_general(oh, x_ref[0], (((0,), (0,)), ((), ())),
                                    preferred_element_type=jnp.float32,
                                    precision=lax.Precision.HIGHEST)
        c_ref[0] += lax.dot_general(oh, jnp.ones((BB, 1), jnp.float32),
                                    (((0,), (0,)), ((), ())),
                                    preferred_element_type=jnp.float32,
                                    precision=lax.Precision.HIGHEST)

    return pl.pallas_call(
        body,
        grid=(2, NBLK),
        in_specs=[
            pl.BlockSpec((1, BB, 16), lambda i, j: (i, j, 0)),
            pl.BlockSpec((1, BB, 1), lambda i, j: (i, j, 0)),
        ],
        out_specs=[
            pl.BlockSpec((1, G, 16), lambda i, j: (i, 0, 0)),
            pl.BlockSpec((1, G, 1), lambda i, j: (i, 0, 0)),
        ],
        out_shape=[
            jax.ShapeDtypeStruct((2, G, 16), jnp.float32),
            jax.ShapeDtypeStruct((2, G, 1), jnp.float32),
        ],
    )(x3, batch)


def _pool_gated(x3, batch, s, cnt, wga, bga):
    def body(x_ref, b_ref, s_ref, c_ref, wg_ref, bg_ref, o_ref):
        j = pl.program_id(1)

        @pl.when(j == 0)
        def _():
            o_ref[...] = jnp.zeros_like(o_ref)

        cmean = s_ref[0] / jnp.maximum(c_ref[0], 1.0)
        cg = jax.nn.sigmoid(
            lax.dot_general(cmean, wg_ref[...], (((1,), (1,)), ((), ())),
                            preferred_element_type=jnp.float32) + bg_ref[...])
        oh = (lax.broadcasted_iota(jnp.int32, (BB, G), 1)
              == b_ref[0]).astype(jnp.float32)
        cn = lax.dot_general(oh, cg, (((1,), (0,)), ((), ())),
                             preferred_element_type=jnp.float32,
                             precision=lax.Precision.HIGHEST)
        gate = jax.nn.sigmoid(jnp.sum(x_ref[0] * cn, axis=1, keepdims=True))
        o_ref[0] += lax.dot_general(oh, gate * x_ref[0],
                                    (((0,), (0,)), ((), ())),
                                    preferred_element_type=jnp.float32,
                                    precision=lax.Precision.HIGHEST)

    return pl.pallas_call(
        body,
        grid=(2, NBLK),
        in_specs=[
            pl.BlockSpec((1, BB, 16), lambda i, j: (i, j, 0)),
            pl.BlockSpec((1, BB, 1), lambda i, j: (i, j, 0)),
            pl.BlockSpec((1, G, 16), lambda i, j: (i, 0, 0)),
            pl.BlockSpec((1, G, 1), lambda i, j: (i, 0, 0)),
            pl.BlockSpec((16, 16), lambda i, j: (0, 0)),
            pl.BlockSpec((1, 16), lambda i, j: (0, 0)),
        ],
        out_specs=pl.BlockSpec((1, G, 16), lambda i, j: (i, 0, 0)),
        out_shape=jax.ShapeDtypeStruct((2, G, 16), jnp.float32),
    )(x3, batch, s, cnt, wga, bga)


def _heads(pooled, wa1, ba1, wa2, ba2, wm, bm, wf1, bf1, wf2, bf2):
    def body(p_ref, wa1_ref, ba1_ref, wa2_ref, ba2_ref, wm_ref, bm_ref,
             wf1_ref, bf1_ref, wf2_ref, bf2_ref, sc_ref, d1_ref, d2_ref):
        hi = p_ref[0]
        hj = p_ref[1]

        def efn(h):
            a = lax.dot_general(h, wa1_ref[...], (((1,), (1,)), ((), ())),
                                preferred_element_type=jnp.float32)
            a = jnp.maximum(a + ba1_ref[...], 0.0)
            a = lax.dot_general(a, wa2_ref[...], (((1,), (1,)), ((), ())),
                                preferred_element_type=jnp.float32)
            a = jnp.tanh(a + ba2_ref[...])
            enc = a * h + h
            m = lax.dot_general(enc, wm_ref[...], (((1,), (1,)), ((), ())),
                                preferred_element_type=jnp.float32)
            return jnp.maximum(m + bm_ref[...], 0.0)

        h_ab = efn(jnp.concatenate([hi, hj], axis=1))
        h_aa = efn(jnp.concatenate([hi, hi], axis=1))
        h_bb = efn(jnp.concatenate([hj, hj], axis=1))
        f = lax.dot_general(h_ab, wf1_ref[...], (((1,), (1,)), ((), ())),
                            preferred_element_type=jnp.float32)
        f = jnp.maximum(f + bf1_ref[...], 0.0)
        f = lax.dot_general(f, wf2_ref[...], (((1,), (1,)), ((), ())),
                            preferred_element_type=jnp.float32)
        sc_ref[...] = f + bf2_ref[...]
        d1_ref[...] = h_ab - h_bb
        d2_ref[...] = h_ab - h_aa

    wf2p = jnp.pad(wf2, ((0, 7), (0, 0)))
    bf2p = jnp.pad(bf2, (0, 7)).reshape(1, 8)
    return pl.pallas_call(
        body,
        out_shape=[
            jax.ShapeDtypeStruct((G, 8), jnp.float32),
            jax.ShapeDtypeStruct((G, 16), jnp.float32),
            jax.ShapeDtypeStruct((G, 16), jnp.float32),
        ],
    )(pooled, wa1, ba1.reshape(1, 8), wa2, ba2.reshape(1, 32),
      wm, bm.reshape(1, 16), wf1, bf1.reshape(1, 8), wf2p, bf2p)


def _pack_edges(src, dst, fill_src):
    perm = jnp.argsort(dst, stable=True)
    src = src[perm]
    dst = dst[perm]
    pad = TOT - E
    src_p = jnp.concatenate(
        [src, jnp.full((pad,), fill_src, jnp.int32)]).reshape(NT * C, CH)
    dst_p = jnp.concatenate(
        [dst, jnp.full((pad,), N, jnp.int32)]).reshape(NT * C, CH)
    return src_p, dst_p


def kernel(x_i, x_j, params, edge_index_i, edge_index_j, batch_i, batch_j):
    p = params
    x = jnp.concatenate([x_i, x_j], axis=0)

    src_i, dst_i = _pack_edges(edge_index_i[0].astype(jnp.int32),
                               edge_index_i[1].astype(jnp.int32), 0)
    src_j, dst_j = _pack_edges(edge_index_j[0].astype(jnp.int32) + N,
                               edge_index_j[1].astype(jnp.int32), N)
    src_all = jnp.concatenate([src_i, src_j], axis=0)
    dst_all = jnp.concatenate([dst_i, dst_j], axis=0)

    batch = jnp.stack([batch_i, batch_j]).astype(jnp.int32).reshape(2, N, 1)

    h = x
    for din, wa, ba, wb, bb, gg, be, eps in [
            (128, p['W1a'], p['b1a'], p['W1b'], p['b1b'], p['g1'], p['be1'],
             p['eps1']),
            (64, p['W2a'], p['b2a'], p['W2b'], p['b2b'], p['g2'], p['be2'],
             p['eps2']),
            (32, p['W3a'], p['b3a'], p['W3b'], p['b3b'], p['g3'], p['be3'],
             p['eps3']),
    ]:
        aggs = []
        for c0 in range(0, din, 64):
            w = min(64, din - c0)
            zrow = jnp.zeros((CH, w), jnp.float32)
            slab = h[:, c0:c0 + w] if din > 64 else h
            a = _make_agg(w)(slab, src_all, dst_all, zrow)
            aggs.append(a.reshape(2, ACC_ROWS, w))
        dout = wb.shape[0]
        scale = jnp.broadcast_to(1.0 + eps, (1, din)).astype(jnp.float32)
        h3 = _dense_layer(h.reshape(2, N, din), aggs,
                          wa, ba.reshape(1, -1), wb, bb.reshape(1, -1),
                          gg.reshape(1, -1), be.reshape(1, -1), scale)
        h = h3.reshape(2 * N, dout)

    x3 = h.reshape(2, N, 16)
    s, cnt = _pool_sums(x3, batch)
    pooled = _pool_gated(x3, batch, s, cnt, p['Wga'],
                         p['bga'].reshape(1, 16))
    score, d1, d2 = _heads(pooled.reshape(2, G, 16),
                           p['Wa1'], p['ba1'], p['Wa2'], p['ba2'],
                           p['Wm'], p['bm'], p['Wf1'], p['bf1'],
                           p['Wf2'], p['bf2'])
    return (score[:, 0], d1, d2)
